# baseline (speedup 1.0000x reference)
import functools

import jax
import jax.numpy as jnp
from jax import lax
from jax.experimental import pallas as pl
from jax.experimental.pallas import tpu as pltpu
from jax.experimental.pallas import tpu_sc as plsc

N, E, D = 10000, 320000, 128

NC = 2
NS = 16
NW = NC * NS
NPAD = 10240
EPW = E // NW
CHUNK = 40
NCHUNK = EPW // CHUNK
RPT = NPAD // NS

DEG_CHUNK = 2000
NDEG_CHUNK = EPW // DEG_CHUNK

_MESH = plsc.VectorSubcoreMesh(core_axis_name="c", subcore_axis_name="s")


@functools.partial(
    pl.kernel,
    mesh=_MESH,
    out_type=jax.ShapeDtypeStruct((NC, NPAD), jnp.float32),
    scratch_types=[
        pltpu.VMEM((DEG_CHUNK,), jnp.int32),
        pltpu.VMEM((DEG_CHUNK,), jnp.float32),
        pltpu.VMEM((RPT,), jnp.float32),
        pltpu.VMEM_SHARED((NPAD,), jnp.float32),
    ],
)
def _sc_degree(e_hbm, deg_hbm, dst_v, ones_v, zero_v, acc):
    cid = lax.axis_index("c")
    sid = lax.axis_index("s")
    wid = sid * NC + cid

    def fill(i, _):
        ones_v[pl.ds(i * 16, 16)] = jnp.full((16,), 1.0, jnp.float32)
        zero_v[pl.ds((i % (RPT // 16)) * 16, 16)] = jnp.zeros((16,), jnp.float32)
        return 0

    lax.fori_loop(0, DEG_CHUNK // 16, fill, 0)

    row0 = pl.multiple_of(sid * RPT, 8)
    pltpu.sync_copy(zero_v, acc.at[pl.ds(row0, RPT)])
    plsc.subcore_barrier()

    base = wid * EPW

    def body(j, _):
        off = pl.multiple_of(E + base + j * DEG_CHUNK, 8)
        pltpu.sync_copy(e_hbm.at[pl.ds(off, DEG_CHUNK)], dst_v)
        pltpu.sync_copy(ones_v, acc.at[dst_v], add=True)
        return 0

    lax.fori_loop(0, NDEG_CHUNK, body, 0)

    plsc.subcore_barrier()
    pltpu.sync_copy(acc.at[pl.ds(row0, RPT)], deg_hbm.at[cid, pl.ds(row0, RPT)])


RING = 5
NBATCH = NCHUNK // RING


@functools.partial(
    pl.kernel,
    mesh=_MESH,
    out_type=jax.ShapeDtypeStruct((NC, NPAD, D), jnp.float32),
    scratch_types=[
        pltpu.VMEM((EPW,), jnp.int32),
        pltpu.VMEM((EPW,), jnp.int32),
        pltpu.VMEM((CHUNK, D), jnp.float32),
        pltpu.VMEM((CHUNK, D), jnp.float32),
        pltpu.VMEM((CHUNK, D), jnp.float32),
        pltpu.VMEM((CHUNK, D), jnp.float32),
        pltpu.VMEM_SHARED((NPAD, D), jnp.float32),
        pltpu.SemaphoreType.DMA((4,)),
        pltpu.SemaphoreType.DMA((4,)),
    ],
)
def _sc_aggregate(g_hbm, e_hbm, out_hbm,
                  srcs, dsts, b0, b1, b2, b3, acc, semg, sems):
    cid = lax.axis_index("c")
    sid = lax.axis_index("s")
    wid = sid * NC + cid
    bufs = (b0, b1, b2, b3)

    def zfill(r, _):
        for c in range(D // 16):
            b0[r, pl.ds(c * 16, 16)] = jnp.zeros((16,), jnp.float32)
        return 0

    lax.fori_loop(0, CHUNK, zfill, 0)
    row0 = pl.multiple_of(sid * RPT, 8)
    for m in range(RPT // CHUNK):
        pltpu.sync_copy(b0, acc.at[pl.ds(row0 + m * CHUNK, CHUNK)])

    base = pl.multiple_of(wid * EPW, 8)
    pltpu.sync_copy(e_hbm.at[pl.ds(base, EPW)], srcs)
    base2 = pl.multiple_of(E + wid * EPW, 8)
    pltpu.sync_copy(e_hbm.at[pl.ds(base2, EPW)], dsts)
    plsc.subcore_barrier()

    def gissue(c, k):
        off = pl.multiple_of(c * CHUNK, 8)
        return pltpu.async_copy(g_hbm.at[srcs.at[pl.ds(off, CHUNK)]],
                                bufs[k], semg.at[k])

    def sissue(c, k):
        off = pl.multiple_of(c * CHUNK, 8)
        return pltpu.async_copy(bufs[k], acc.at[dsts.at[pl.ds(off, CHUNK)]],
                                sems.at[k], add=True)

    gissue(0, 0).wait()
    gissue(1, 1).wait()

    def body(u, _):
        c = u * 4
        sa = (sissue(c + 0, 0), sissue(c + 1, 1))
        gb = (gissue(c + 2, 2), gissue(c + 3, 3))
        sa[0].wait()
        sa[1].wait()
        gb[0].wait()
        gb[1].wait()
        sb = (sissue(c + 2, 2), sissue(c + 3, 3))
        ga = (gissue(c + 4, 0), gissue(c + 5, 1))
        sb[0].wait()
        sb[1].wait()
        ga[0].wait()
        ga[1].wait()
        return 0

    lax.fori_loop(0, (NCHUNK - 2) // 4, body, 0)

    sissue(NCHUNK - 2, 0).wait()
    sissue(NCHUNK - 1, 1).wait()

    plsc.subcore_barrier()
    pltpu.sync_copy(acc.at[pl.ds(row0, RPT)], out_hbm.at[cid, pl.ds(row0, RPT)])


_RB = 1024


def _prep_body(x_ref, w_ref, deg_ref, g_ref):
    h = lax.dot_general(
        x_ref[...], w_ref[...], (((1,), (1,)), ((), ())),
        preferred_element_type=jnp.float32)
    deg = deg_ref[0, :] + deg_ref[1, :] + 1.0
    dis = lax.rsqrt(deg)
    g_ref[...] = h * dis[:, None]


_prep = pl.pallas_call(
    _prep_body,
    grid=(pl.cdiv(N, _RB),),
    in_specs=[
        pl.BlockSpec((_RB, D), lambda i: (i, 0)),
        pl.BlockSpec((D, D), lambda i: (0, 0)),
        pl.BlockSpec((2, _RB), lambda i: (0, i)),
    ],
    out_specs=pl.BlockSpec((_RB, D), lambda i: (i, 0)),
    out_shape=jax.ShapeDtypeStruct((N, D), jnp.float32),
)


def _final_body(p_ref, g_ref, deg_ref, b_ref, o_ref):
    deg = deg_ref[0, :] + deg_ref[1, :] + 1.0
    dis = lax.rsqrt(deg)
    s = p_ref[0] + p_ref[1] + g_ref[...]
    o_ref[...] = s * dis[:, None] + b_ref[...][None, :]


_final = pl.pallas_call(
    _final_body,
    grid=(pl.cdiv(N, _RB),),
    in_specs=[
        pl.BlockSpec((2, _RB, D), lambda i: (0, i, 0)),
        pl.BlockSpec((_RB, D), lambda i: (i, 0)),
        pl.BlockSpec((2, _RB), lambda i: (0, i)),
        pl.BlockSpec((D,), lambda i: (0,)),
    ],
    out_specs=pl.BlockSpec((_RB, D), lambda i: (i, 0)),
    out_shape=jax.ShapeDtypeStruct((N, D), jnp.float32),
)


def kernel(x, edge_index, W, b):
    ei = edge_index.reshape(2 * E)
    deg_p = _sc_degree(ei)
    g = _prep(x, W, deg_p)
    p = _sc_aggregate(g, ei)
    return _final(p, g, deg_p, b)


# speedup vs baseline: 1.3286x; 1.3286x over previous
import functools

import jax
import jax.numpy as jnp
from jax import lax
from jax.experimental import pallas as pl
from jax.experimental.pallas import tpu as pltpu
from jax.experimental.pallas import tpu_sc as plsc

N, E, D = 10000, 320000, 128

NC = 2
NS = 16
NW = NC * NS
NPAD = 10240
EPW = E // NW
CHUNK = 40
NCHUNK = EPW // CHUNK
RPT = NPAD // NS

DEG_CHUNK = 2000
NDEG_CHUNK = EPW // DEG_CHUNK

_MESH = plsc.VectorSubcoreMesh(core_axis_name="c", subcore_axis_name="s")


@functools.partial(
    pl.kernel,
    mesh=_MESH,
    out_type=jax.ShapeDtypeStruct((NC, NPAD), jnp.float32),
    scratch_types=[
        pltpu.VMEM((DEG_CHUNK,), jnp.int32),
        pltpu.VMEM((DEG_CHUNK,), jnp.float32),
        pltpu.VMEM((RPT,), jnp.float32),
        pltpu.VMEM_SHARED((NPAD,), jnp.float32),
    ],
)
def _sc_degree(e_hbm, deg_hbm, dst_v, ones_v, zero_v, acc):
    cid = lax.axis_index("c")
    sid = lax.axis_index("s")
    wid = sid * NC + cid

    def fill(i, _):
        ones_v[pl.ds(i * 16, 16)] = jnp.full((16,), 1.0, jnp.float32)
        zero_v[pl.ds((i % (RPT // 16)) * 16, 16)] = jnp.zeros((16,), jnp.float32)
        return 0

    lax.fori_loop(0, DEG_CHUNK // 16, fill, 0)

    row0 = pl.multiple_of(sid * RPT, 8)
    pltpu.sync_copy(zero_v, acc.at[pl.ds(row0, RPT)])
    plsc.subcore_barrier()

    base = wid * EPW

    def body(j, _):
        off = pl.multiple_of(E + base + j * DEG_CHUNK, 8)
        pltpu.sync_copy(e_hbm.at[pl.ds(off, DEG_CHUNK)], dst_v)
        pltpu.sync_copy(ones_v, acc.at[dst_v], add=True)
        return 0

    lax.fori_loop(0, NDEG_CHUNK, body, 0)

    plsc.subcore_barrier()
    pltpu.sync_copy(acc.at[pl.ds(row0, RPT)], deg_hbm.at[cid, pl.ds(row0, RPT)])


RING = 5
NBATCH = NCHUNK // RING


@functools.partial(
    pl.kernel,
    mesh=_MESH,
    out_type=jax.ShapeDtypeStruct((NC, NPAD, D), jnp.float32),
    scratch_types=[
        pltpu.VMEM((EPW,), jnp.int32),
        pltpu.VMEM((EPW,), jnp.int32),
        pltpu.VMEM((CHUNK, D), jnp.float32),
        pltpu.VMEM((CHUNK, D), jnp.float32),
        pltpu.VMEM((CHUNK, D), jnp.float32),
        pltpu.VMEM((CHUNK, D), jnp.float32),
        pltpu.VMEM((CHUNK, D), jnp.float32),
        pltpu.VMEM_SHARED((NPAD, D), jnp.float32),
        pltpu.SemaphoreType.DMA,
        pltpu.SemaphoreType.DMA,
        pltpu.SemaphoreType.DMA,
        pltpu.SemaphoreType.DMA,
        pltpu.SemaphoreType.DMA,
        pltpu.SemaphoreType.DMA,
        pltpu.SemaphoreType.DMA,
        pltpu.SemaphoreType.DMA,
        pltpu.SemaphoreType.DMA,
        pltpu.SemaphoreType.DMA,
    ],
)
def _sc_aggregate(g_hbm, e_hbm, out_hbm,
                  srcs, dsts, b0, b1, b2, b3, b4, acc,
                  sg0, sg1, sg2, sg3, sg4, ss0, ss1, ss2, ss3, ss4):
    cid = lax.axis_index("c")
    sid = lax.axis_index("s")
    wid = sid * NC + cid
    bufs = (b0, b1, b2, b3, b4)
    semg = (sg0, sg1, sg2, sg3, sg4)
    sems = (ss0, ss1, ss2, ss3, ss4)

    def zfill(r, _):
        for c in range(D // 16):
            b0[r, pl.ds(c * 16, 16)] = jnp.zeros((16,), jnp.float32)
        return 0

    lax.fori_loop(0, CHUNK, zfill, 0)
    row0 = pl.multiple_of(sid * RPT, 8)
    for m in range(RPT // CHUNK):
        pltpu.sync_copy(b0, acc.at[pl.ds(row0 + m * CHUNK, CHUNK)])

    base = pl.multiple_of(wid * EPW, 8)
    pltpu.sync_copy(e_hbm.at[pl.ds(base, EPW)], srcs)
    base2 = pl.multiple_of(E + wid * EPW, 8)
    pltpu.sync_copy(e_hbm.at[pl.ds(base2, EPW)], dsts)
    plsc.subcore_barrier()

    def gissue(c, k):
        off = pl.multiple_of(c * CHUNK, 8)
        pltpu.async_copy(g_hbm.at[srcs.at[pl.ds(off, CHUNK)]], bufs[k],
                         semg[k])

    def gwait(k):
        pltpu.make_async_copy(g_hbm.at[srcs.at[pl.ds(0, CHUNK)]], bufs[k],
                              semg[k]).wait()

    def sissue(c, k):
        off = pl.multiple_of(c * CHUNK, 8)
        pltpu.async_copy(bufs[k], acc.at[dsts.at[pl.ds(off, CHUNK)]],
                         sems[k], add=True)

    def swait(k):
        pltpu.make_async_copy(bufs[k], acc.at[dsts.at[pl.ds(0, CHUNK)]],
                              sems[k]).wait()

    LAG = 3
    for k in range(RING):
        gissue(k, k)
    for j in range(RING - LAG):
        gwait(j)
        sissue(j, j)

    def body(t, _):
        for k in range(RING):
            c = t * RING + k
            swait(k)
            gissue(c, k)
            c2 = c - LAG
            k2 = (k + RING - LAG) % RING
            gwait(k2)
            sissue(c2, k2)
        return 0

    lax.fori_loop(1, NBATCH, body, 0)

    for c2 in range(NCHUNK - LAG, NCHUNK):
        k2 = c2 % RING
        gwait(k2)
        sissue(c2, k2)
    for m in range(NCHUNK - RING, NCHUNK):
        swait(m % RING)

    plsc.subcore_barrier()
    pltpu.sync_copy(acc.at[pl.ds(row0, RPT)], out_hbm.at[cid, pl.ds(row0, RPT)])


_RB = 1024


def _prep_body(x_ref, w_ref, deg_ref, g_ref):
    h = lax.dot_general(
        x_ref[...], w_ref[...], (((1,), (1,)), ((), ())),
        preferred_element_type=jnp.float32)
    deg = deg_ref[0, :] + deg_ref[1, :] + 1.0
    dis = lax.rsqrt(deg)
    g_ref[...] = h * dis[:, None]


_prep = pl.pallas_call(
    _prep_body,
    grid=(pl.cdiv(N, _RB),),
    in_specs=[
        pl.BlockSpec((_RB, D), lambda i: (i, 0)),
        pl.BlockSpec((D, D), lambda i: (0, 0)),
        pl.BlockSpec((2, _RB), lambda i: (0, i)),
    ],
    out_specs=pl.BlockSpec((_RB, D), lambda i: (i, 0)),
    out_shape=jax.ShapeDtypeStruct((N, D), jnp.float32),
)


def _final_body(p_ref, g_ref, deg_ref, b_ref, o_ref):
    deg = deg_ref[0, :] + deg_ref[1, :] + 1.0
    dis = lax.rsqrt(deg)
    s = p_ref[0] + p_ref[1] + g_ref[...]
    o_ref[...] = s * dis[:, None] + b_ref[...][None, :]


_final = pl.pallas_call(
    _final_body,
    grid=(pl.cdiv(N, _RB),),
    in_specs=[
        pl.BlockSpec((2, _RB, D), lambda i: (0, i, 0)),
        pl.BlockSpec((_RB, D), lambda i: (i, 0)),
        pl.BlockSpec((2, _RB), lambda i: (0, i)),
        pl.BlockSpec((D,), lambda i: (0,)),
    ],
    out_specs=pl.BlockSpec((_RB, D), lambda i: (i, 0)),
    out_shape=jax.ShapeDtypeStruct((N, D), jnp.float32),
)


def kernel(x, edge_index, W, b):
    ei = edge_index.reshape(2 * E)
    deg_p = _sc_degree(ei)
    g = _prep(x, W, deg_p)
    p = _sc_aggregate(g, ei)
    return _final(p, g, deg_p, b)
